# baseline (speedup 1.0000x reference)
import functools

import jax
import jax.numpy as jnp
from jax import lax
from jax.experimental import pallas as pl
from jax.experimental.pallas import tpu as pltpu
from jax.experimental.pallas import tpu_sc as plsc

_NUM_SAMPLES = 32


def _build_sampler(B, N, D, S):
    info = plsc.get_sparse_core_info()
    NC, NS, L = info.num_cores, info.num_subcores, info.num_lanes
    NW = NC * NS
    TW = 128
    assert B % L == 0 and D % L == 0 and S % L == 0
    NT_FULL = N // TW
    PW = N - NT_FULL * TW
    JMAX = -(-NT_FULL // NW)
    RING = 256
    HALF = RING // 2
    _HALF_SHIFT = HALF.bit_length() - 1

    mesh = plsc.VectorSubcoreMesh(core_axis_name="c", subcore_axis_name="s")

    scratch = [
        pltpu.VMEM((B + L,), jnp.int32),
        pltpu.VMEM((B + L,), jnp.float32),
        pltpu.VMEM((B + L,), jnp.int32),
        pltpu.VMEM((B + L,), jnp.int32),
        pltpu.VMEM((D, TW), jnp.int32),
        pltpu.VMEM((D, TW), jnp.int32),
        pltpu.VMEM((D, TW), jnp.float32),
        pltpu.VMEM((D, TW), jnp.float32),
        pltpu.VMEM((RING * S,), jnp.int32),
        pltpu.VMEM((RING * S,), jnp.float32),
        pltpu.SemaphoreType.DMA,
        pltpu.SemaphoreType.DMA,
        pltpu.SemaphoreType.DMA,
        pltpu.SemaphoreType.DMA,
    ]
    if PW:
        scratch += [
            pltpu.VMEM((PW, D), jnp.int32),
            pltpu.VMEM((PW, D), jnp.float32),
            pltpu.SemaphoreType.DMA,
        ]

    @functools.partial(
        pl.kernel,
        mesh=mesh,
        compiler_params=pltpu.CompilerParams(needs_layout_passes=False),
        out_type=(
            jax.ShapeDtypeStruct((B * S,), jnp.int32),
            jax.ShapeDtypeStruct((B * S,), jnp.float32),
        ),
        scratch_types=scratch,
    )
    def sampler(ids_hbm, tss_hbm, adjT_hbm, tsT_hbm, *rest):
        if PW:
            (adj_tl_hbm, ts_tl_hbm, out_n_hbm, out_t_hbm,
             ids_all, tss_all, wl, bwl, adj_b0, adj_b1, ts_b0, ts_b1,
             sn, st, sem_b0, sem_b1, sem_on, sem_ot,
             adj_tl, ts_tl, sem_tl) = rest
        else:
            (out_n_hbm, out_t_hbm,
             ids_all, tss_all, wl, bwl, adj_b0, adj_b1, ts_b0, ts_b1,
             sn, st, sem_b0, sem_b1, sem_on, sem_ot) = rest
        wid = lax.axis_index("s") * NC + lax.axis_index("c")
        lanes = lax.iota(jnp.int32, L)

        pltpu.sync_copy(ids_hbm, ids_all.at[pl.ds(0, B)])
        pltpu.sync_copy(tss_hbm, tss_all.at[pl.ds(0, B)])

        if PW:
            pltpu.async_copy(adj_tl_hbm, adj_tl, sem_tl)
            pltpu.async_copy(ts_tl_hbm, ts_tl, sem_tl)


        def detect(g, nh):
            qv = ids_all[pl.ds(g * L, L)]
            m = ((qv >> 7) & (NW - 1)) == wid
            plsc.store_compressed(wl.at[pl.ds(nh, L)], g * L + lanes, mask=m)
            return nh + plsc.all_reduce_population_count(m)[0]

        nh = lax.fori_loop(0, B // L, detect, jnp.int32(0))
        ng = (nh + L - 1) >> 4

        def fire_block(t, adj_b, ts_b, sem_b):
            c0 = pl.multiple_of(t * TW, TW)
            pltpu.async_copy(adjT_hbm.at[:, pl.ds(c0, TW)], adj_b, sem_b)
            pltpu.async_copy(tsT_hbm.at[:, pl.ds(c0, TW)], ts_b, sem_b)

        def wait_block(adj_b, ts_b, sem_b):
            pltpu.make_async_copy(
                adjT_hbm.at[:, pl.ds(0, TW)], adj_b, sem_b).wait()
            pltpu.make_async_copy(
                tsT_hbm.at[:, pl.ds(0, TW)], ts_b, sem_b).wait()

        def drain_one_(i, c):
            pltpu.make_async_copy(
                sn.at[pl.ds(0, S)], out_n_hbm.at[pl.ds(0, S)], sem_on).wait()
            pltpu.make_async_copy(
                st.at[pl.ds(0, S)], out_t_hbm.at[pl.ds(0, S)], sem_ot).wait()
            return c

        def process_block(t_eff, adj_b, ts_b, sbase):
            def scan(g, nb):
                wv = wl[pl.ds(g * L, L)]
                wv = wv & (B - 1)
                idv = plsc.load_gather(ids_all, [wv])
                m = ((idv >> 7) == t_eff) & ((g * L + lanes) < nh)
                plsc.store_compressed(bwl.at[pl.ds(nb, L)], wv, mask=m)
                return nb + plsc.all_reduce_population_count(m)[0]

            nb = lax.fori_loop(0, ng, scan, jnp.int32(0))

            def hit(i, base_h):
                slot = sbase + i
                qx = bwl[pl.ds(base_h + i, L)][0]
                qid = ids_all[pl.ds(qx, L)][0]
                tq = tss_all[pl.ds(qx, L)][0]
                cs = jnp.full((L,), qid & (TW - 1), jnp.int32)
                acc = jnp.zeros((L,), jnp.int32)
                for k in range(D // L):
                    v = plsc.load_gather(ts_b, [k * L + lanes, cs])
                    acc = acc + (v < tq).astype(jnp.int32)
                lo = jnp.sum(acc) - S
                for h in range(S // L):
                    rows = lo + h * L + lanes
                    nv = plsc.load_gather(adj_b, [rows, cs])
                    tv = plsc.load_gather(ts_b, [rows, cs])
                    sn[pl.ds(slot * S + h * L, L)] = nv
                    st[pl.ds(slot * S + h * L, L)] = tv
                pltpu.async_copy(
                    sn.at[pl.ds(slot * S, S)], out_n_hbm.at[pl.ds(qx * S, S)],
                    sem_on)
                pltpu.async_copy(
                    st.at[pl.ds(slot * S, S)], out_t_hbm.at[pl.ds(qx * S, S)],
                    sem_ot)
                return base_h

            def wave(w, prev_cnt):
                lax.fori_loop(0, prev_cnt, drain_one_, 0)
                base_h = w * HALF
                cnt_w = jnp.minimum(nb - base_h, HALF)
                lax.fori_loop(0, cnt_w, hit, base_h)
                return cnt_w

            nwaves = (nb + HALF - 1) >> _HALF_SHIFT
            return lax.fori_loop(0, nwaves, wave, jnp.int32(0))

        bufs = [(adj_b0, ts_b0, sem_b0), (adj_b1, ts_b1, sem_b1)]
        fire_block(wid, *bufs[0])

        pend = [jnp.int32(0), jnp.int32(0)]
        for j in range(JMAX):
            tj = wid + NW * j
            if j + 1 < JMAX:
                tn = jnp.minimum(wid + NW * (j + 1), NT_FULL - 1)
                fire_block(tn, *bufs[(j + 1) % 2])

            wait_block(*bufs[j % 2])
            lax.fori_loop(0, pend[j % 2], drain_one_, 0)
            pend[j % 2] = process_block(
                tj, bufs[j % 2][0], bufs[j % 2][1], (j % 2) * HALF)
        lax.fori_loop(0, pend[0], drain_one_, 0)
        lax.fori_loop(0, pend[1], drain_one_, 0)

        if PW:
            pltpu.make_async_copy(adj_tl_hbm, adj_tl, sem_tl).wait()
            pltpu.make_async_copy(ts_tl_hbm, ts_tl, sem_tl).wait()

            def scan_tl(g, nb):
                wv = wl[pl.ds(g * L, L)] & (B - 1)
                idv = plsc.load_gather(ids_all, [wv])
                m = ((idv >> 7) == NT_FULL) & ((g * L + lanes) < nh)
                plsc.store_compressed(bwl.at[pl.ds(nb, L)], wv, mask=m)
                return nb + plsc.all_reduce_population_count(m)[0]

            nb_tl = lax.fori_loop(0, ng, scan_tl, jnp.int32(0))

            def hit_tl(i, base_h):
                qx = bwl[pl.ds(base_h + i, L)][0]
                qid = ids_all[pl.ds(qx, L)][0]
                tq = tss_all[pl.ds(qx, L)][0]
                rs = jnp.full((L,), qid - NT_FULL * TW, jnp.int32)
                acc = jnp.zeros((L,), jnp.int32)
                for k in range(D // L):
                    v = plsc.load_gather(ts_tl, [rs, k * L + lanes])
                    acc = acc + (v < tq).astype(jnp.int32)
                lo = jnp.sum(acc) - S
                for h in range(S // L):
                    cols = lo + h * L + lanes
                    sn[pl.ds(i * S + h * L, L)] = plsc.load_gather(adj_tl, [rs, cols])
                    st[pl.ds(i * S + h * L, L)] = plsc.load_gather(ts_tl, [rs, cols])
                pltpu.async_copy(
                    sn.at[pl.ds(i * S, S)], out_n_hbm.at[pl.ds(qx * S, S)],
                    sem_on)
                pltpu.async_copy(
                    st.at[pl.ds(i * S, S)], out_t_hbm.at[pl.ds(qx * S, S)],
                    sem_ot)
                return base_h

            def wave_tl(w, prev_cnt):
                lax.fori_loop(0, prev_cnt, drain_one_, 0)
                base_h = w * HALF
                cnt_w = jnp.minimum(nb_tl - base_h, HALF)
                lax.fori_loop(0, cnt_w, hit_tl, base_h)
                return cnt_w

            last_tl = lax.fori_loop(
                0, (nb_tl + HALF - 1) >> _HALF_SHIFT, wave_tl, jnp.int32(0))
            lax.fori_loop(0, last_tl, drain_one_, 0)

    return sampler


def kernel(ids, tss, batch_size, num_samples, adj_info, ts_info):
    B = ids.shape[0]
    N, D = adj_info.shape
    S = _NUM_SAMPLES
    sampler = _build_sampler(B, N, D, S)
    ntail = N % 128
    if ntail:
        args = (ids, tss, adj_info.T, ts_info.T,
                adj_info[N - ntail:, :], ts_info[N - ntail:, :])
    else:
        args = (ids, tss, adj_info.T, ts_info.T)
    out_n, out_t = sampler(*args)
    return out_n, out_t


# speedup vs baseline: 1.0625x; 1.0625x over previous
import functools

import jax
import jax.numpy as jnp
from jax import lax
from jax.experimental import pallas as pl
from jax.experimental.pallas import tpu as pltpu
from jax.experimental.pallas import tpu_sc as plsc

_NUM_SAMPLES = 32


def _build_sampler(B, N, D, S):
    info = plsc.get_sparse_core_info()
    NC, NS, L = info.num_cores, info.num_subcores, info.num_lanes
    NW = NC * NS
    TW = 128
    assert B % L == 0 and D % L == 0 and S % L == 0
    NT_FULL = N // TW
    PW = N - NT_FULL * TW
    JMAX = -(-NT_FULL // NW)
    RING = 256
    HALF = RING // 2
    _HALF_SHIFT = HALF.bit_length() - 1

    mesh = plsc.VectorSubcoreMesh(core_axis_name="c", subcore_axis_name="s")

    scratch = [
        pltpu.VMEM((B + L,), jnp.int32),
        pltpu.VMEM((B + L,), jnp.float32),
        pltpu.VMEM((B + L,), jnp.int32),
        pltpu.VMEM((B + L,), jnp.int32),
        pltpu.VMEM((D, TW), jnp.int32),
        pltpu.VMEM((D, TW), jnp.int32),
        pltpu.VMEM((D, TW), jnp.int32),
        pltpu.VMEM((D, TW), jnp.int32),
        pltpu.VMEM((D, TW), jnp.float32),
        pltpu.VMEM((D, TW), jnp.float32),
        pltpu.VMEM((D, TW), jnp.float32),
        pltpu.VMEM((D, TW), jnp.float32),
        pltpu.VMEM((RING * S,), jnp.int32),
        pltpu.VMEM((RING * S,), jnp.float32),
        pltpu.SemaphoreType.DMA,
        pltpu.SemaphoreType.DMA,
        pltpu.SemaphoreType.DMA,
        pltpu.SemaphoreType.DMA,
        pltpu.SemaphoreType.DMA,
        pltpu.SemaphoreType.DMA,
    ]
    if PW:
        scratch += [
            pltpu.VMEM((PW, D), jnp.int32),
            pltpu.VMEM((PW, D), jnp.float32),
            pltpu.SemaphoreType.DMA,
        ]

    @functools.partial(
        pl.kernel,
        mesh=mesh,
        compiler_params=pltpu.CompilerParams(needs_layout_passes=False),
        out_type=(
            jax.ShapeDtypeStruct((B * S,), jnp.int32),
            jax.ShapeDtypeStruct((B * S,), jnp.float32),
        ),
        scratch_types=scratch,
    )
    def sampler(ids_hbm, tss_hbm, adjT_hbm, tsT_hbm, *rest):
        if PW:
            (adj_tl_hbm, ts_tl_hbm, out_n_hbm, out_t_hbm,
             ids_all, tss_all, wl, bwl,
             adj_b0, adj_b1, adj_b2, adj_b3, ts_b0, ts_b1, ts_b2, ts_b3,
             sn, st, sem_b0, sem_b1, sem_b2, sem_b3, sem_on, sem_ot,
             adj_tl, ts_tl, sem_tl) = rest
        else:
            (out_n_hbm, out_t_hbm,
             ids_all, tss_all, wl, bwl,
             adj_b0, adj_b1, adj_b2, adj_b3, ts_b0, ts_b1, ts_b2, ts_b3,
             sn, st, sem_b0, sem_b1, sem_b2, sem_b3, sem_on, sem_ot) = rest
        wid = lax.axis_index("s") * NC + lax.axis_index("c")
        lanes = lax.iota(jnp.int32, L)

        pltpu.sync_copy(ids_hbm, ids_all.at[pl.ds(0, B)])
        pltpu.sync_copy(tss_hbm, tss_all.at[pl.ds(0, B)])

        if PW:
            pltpu.async_copy(adj_tl_hbm, adj_tl, sem_tl)
            pltpu.async_copy(ts_tl_hbm, ts_tl, sem_tl)


        def detect(g, nh):
            qv = ids_all[pl.ds(g * L, L)]
            m = ((qv >> 7) & (NW - 1)) == wid
            plsc.store_compressed(wl.at[pl.ds(nh, L)], g * L + lanes, mask=m)
            return nh + plsc.all_reduce_population_count(m)[0]

        nh = lax.fori_loop(0, B // L, detect, jnp.int32(0))
        ng = (nh + L - 1) >> 4

        def fire_block(t, adj_b, ts_b, sem_b):
            c0 = pl.multiple_of(t * TW, TW)
            pltpu.async_copy(adjT_hbm.at[:, pl.ds(c0, TW)], adj_b, sem_b)
            pltpu.async_copy(tsT_hbm.at[:, pl.ds(c0, TW)], ts_b, sem_b)

        def wait_block(adj_b, ts_b, sem_b):
            pltpu.make_async_copy(
                adjT_hbm.at[:, pl.ds(0, TW)], adj_b, sem_b).wait()
            pltpu.make_async_copy(
                tsT_hbm.at[:, pl.ds(0, TW)], ts_b, sem_b).wait()

        def drain_one_(i, c):
            pltpu.make_async_copy(
                sn.at[pl.ds(0, S)], out_n_hbm.at[pl.ds(0, S)], sem_on).wait()
            pltpu.make_async_copy(
                st.at[pl.ds(0, S)], out_t_hbm.at[pl.ds(0, S)], sem_ot).wait()
            return c

        def process_block(t_eff, adj_b, ts_b, sbase):
            def scan(g, nb):
                wv = wl[pl.ds(g * L, L)]
                wv = wv & (B - 1)
                idv = plsc.load_gather(ids_all, [wv])
                m = ((idv >> 7) == t_eff) & ((g * L + lanes) < nh)
                plsc.store_compressed(bwl.at[pl.ds(nb, L)], wv, mask=m)
                return nb + plsc.all_reduce_population_count(m)[0]

            nb = lax.fori_loop(0, ng, scan, jnp.int32(0))

            def hit(i, base_h):
                slot = sbase + i
                qx = bwl[pl.ds(base_h + i, L)][0]
                qid = ids_all[pl.ds(qx, L)][0]
                tq = tss_all[pl.ds(qx, L)][0]
                cs = jnp.full((L,), qid & (TW - 1), jnp.int32)
                acc = jnp.zeros((L,), jnp.int32)
                for k in range(D // L):
                    v = plsc.load_gather(ts_b, [k * L + lanes, cs])
                    acc = acc + (v < tq).astype(jnp.int32)
                lo = jnp.sum(acc) - S
                for h in range(S // L):
                    rows = lo + h * L + lanes
                    nv = plsc.load_gather(adj_b, [rows, cs])
                    tv = plsc.load_gather(ts_b, [rows, cs])
                    sn[pl.ds(slot * S + h * L, L)] = nv
                    st[pl.ds(slot * S + h * L, L)] = tv
                pltpu.async_copy(
                    sn.at[pl.ds(slot * S, S)], out_n_hbm.at[pl.ds(qx * S, S)],
                    sem_on)
                pltpu.async_copy(
                    st.at[pl.ds(slot * S, S)], out_t_hbm.at[pl.ds(qx * S, S)],
                    sem_ot)
                return base_h

            def wave(w, prev_cnt):
                lax.fori_loop(0, prev_cnt, drain_one_, 0)
                base_h = w * HALF
                cnt_w = jnp.minimum(nb - base_h, HALF)
                lax.fori_loop(0, cnt_w, hit, base_h)
                return cnt_w

            nwaves = (nb + HALF - 1) >> _HALF_SHIFT
            return lax.fori_loop(0, nwaves, wave, jnp.int32(0))

        bufs = [(adj_b0, ts_b0, sem_b0), (adj_b1, ts_b1, sem_b1),
                (adj_b2, ts_b2, sem_b2), (adj_b3, ts_b3, sem_b3)]
        NBUF = len(bufs)
        for jp in range(min(NBUF - 1, JMAX)):
            tp = jnp.minimum(wid + NW * jp, NT_FULL - 1)
            fire_block(tp, *bufs[jp % NBUF])

        pend = [jnp.int32(0), jnp.int32(0)]
        for j in range(JMAX):
            tj = wid + NW * j
            if j + NBUF - 1 < JMAX:
                tn = jnp.minimum(wid + NW * (j + NBUF - 1), NT_FULL - 1)
                fire_block(tn, *bufs[(j + NBUF - 1) % NBUF])

            wait_block(*bufs[j % NBUF])
            lax.fori_loop(0, pend[j % 2], drain_one_, 0)
            pend[j % 2] = process_block(
                tj, bufs[j % NBUF][0], bufs[j % NBUF][1], (j % 2) * HALF)
        lax.fori_loop(0, pend[0], drain_one_, 0)
        lax.fori_loop(0, pend[1], drain_one_, 0)

        if PW:
            pltpu.make_async_copy(adj_tl_hbm, adj_tl, sem_tl).wait()
            pltpu.make_async_copy(ts_tl_hbm, ts_tl, sem_tl).wait()

            def scan_tl(g, nb):
                wv = wl[pl.ds(g * L, L)] & (B - 1)
                idv = plsc.load_gather(ids_all, [wv])
                m = ((idv >> 7) == NT_FULL) & ((g * L + lanes) < nh)
                plsc.store_compressed(bwl.at[pl.ds(nb, L)], wv, mask=m)
                return nb + plsc.all_reduce_population_count(m)[0]

            nb_tl = lax.fori_loop(0, ng, scan_tl, jnp.int32(0))

            def hit_tl(i, base_h):
                qx = bwl[pl.ds(base_h + i, L)][0]
                qid = ids_all[pl.ds(qx, L)][0]
                tq = tss_all[pl.ds(qx, L)][0]
                rs = jnp.full((L,), qid - NT_FULL * TW, jnp.int32)
                acc = jnp.zeros((L,), jnp.int32)
                for k in range(D // L):
                    v = plsc.load_gather(ts_tl, [rs, k * L + lanes])
                    acc = acc + (v < tq).astype(jnp.int32)
                lo = jnp.sum(acc) - S
                for h in range(S // L):
                    cols = lo + h * L + lanes
                    sn[pl.ds(i * S + h * L, L)] = plsc.load_gather(adj_tl, [rs, cols])
                    st[pl.ds(i * S + h * L, L)] = plsc.load_gather(ts_tl, [rs, cols])
                pltpu.async_copy(
                    sn.at[pl.ds(i * S, S)], out_n_hbm.at[pl.ds(qx * S, S)],
                    sem_on)
                pltpu.async_copy(
                    st.at[pl.ds(i * S, S)], out_t_hbm.at[pl.ds(qx * S, S)],
                    sem_ot)
                return base_h

            def wave_tl(w, prev_cnt):
                lax.fori_loop(0, prev_cnt, drain_one_, 0)
                base_h = w * HALF
                cnt_w = jnp.minimum(nb_tl - base_h, HALF)
                lax.fori_loop(0, cnt_w, hit_tl, base_h)
                return cnt_w

            last_tl = lax.fori_loop(
                0, (nb_tl + HALF - 1) >> _HALF_SHIFT, wave_tl, jnp.int32(0))
            lax.fori_loop(0, last_tl, drain_one_, 0)

    return sampler


def kernel(ids, tss, batch_size, num_samples, adj_info, ts_info):
    B = ids.shape[0]
    N, D = adj_info.shape
    S = _NUM_SAMPLES
    sampler = _build_sampler(B, N, D, S)
    ntail = N % 128
    if ntail:
        args = (ids, tss, adj_info.T, ts_info.T,
                adj_info[N - ntail:, :], ts_info[N - ntail:, :])
    else:
        args = (ids, tss, adj_info.T, ts_info.T)
    out_n, out_t = sampler(*args)
    return out_n, out_t
